# baseline (speedup 1.0000x reference)
import functools

import jax
import jax.numpy as jnp
from jax import lax
from jax.experimental import pallas as pl
from jax.experimental.pallas import tpu as pltpu
from jax.experimental.pallas import tpu_sc as plsc

EPS = 1e-16

N_SC = 4096
SC_CHUNK = 1024


def _cos_body(memt_ref, k_ref, cos_ref):
    memt = memt_ref[...]
    kk = k_ref[...] + EPS
    num = jnp.sum(memt * kk[:, :, None], axis=1)
    ssq = jnp.sum(memt * memt, axis=1)
    normk = jnp.sqrt(jnp.sum(kk * kk, axis=-1))
    denom = jnp.sqrt(ssq) * normk[:, None]
    cos_ref[...] = num / jnp.maximum(denom, 1e-8)


def _w_body(cos_ref, num_ref, ssq_ref, k_ref, wprev_ref, beta_ref, g_ref,
            s_ref, gamma_ref, w_ref):
    kk = k_ref[...] + EPS
    normk = jnp.sqrt(jnp.sum(kk * kk, axis=-1, keepdims=True))
    denom = jnp.maximum(jnp.sqrt(ssq_ref[...]) * normk, 1e-8)
    cos_sc = num_ref[...] / denom
    cos = jnp.concatenate([cos_ref[...], cos_sc], axis=1)
    beta = beta_ref[...]
    x = beta * cos
    x = x - jnp.max(x, axis=1, keepdims=True)
    ex = jnp.exp(x)
    wc = ex / jnp.sum(ex, axis=1, keepdims=True)
    g = g_ref[...]
    wg = g * wc + (1.0 - g) * wprev_ref[...]
    s = s_ref[...]
    left = jnp.concatenate([wg[:, -1:], wg[:, :-1]], axis=1)
    right = jnp.concatenate([wg[:, 1:], wg[:, :1]], axis=1)
    sh = left * s[:, 0:1] + wg * s[:, 1:2] + right * s[:, 2:3]
    gamma = gamma_ref[...]
    wpow = jnp.where(sh > 0.0,
                     jnp.exp(gamma * jnp.log(jnp.maximum(sh, 1e-38))),
                     0.0)
    w_ref[...] = wpow / (jnp.sum(wpow, axis=1, keepdims=True) + EPS)


def _read_body(w_ref, memt_ref, out_ref):
    @pl.when(pl.program_id(0) == 0)
    def _():
        out_ref[...] = jnp.zeros_like(out_ref)

    w = w_ref[...]
    memt = memt_ref[...]
    out_ref[...] += jnp.sum(memt * w[:, None, :], axis=2)


def _make_sc_cos(B, M, N, n_tc, ch):
    n_sc = N - n_tc
    nch = n_sc // ch
    mesh = plsc.VectorSubcoreMesh(core_axis_name="c", subcore_axis_name="s")

    @functools.partial(
        pl.kernel,
        out_type=(jax.ShapeDtypeStruct((B, n_sc), jnp.float32),
                  jax.ShapeDtypeStruct((B, n_sc), jnp.float32)),
        mesh=mesh,
        scratch_types=[
            pltpu.VMEM((M, ch), jnp.float32),
            pltpu.VMEM((M,), jnp.float32),
            pltpu.VMEM((ch,), jnp.float32),
            pltpu.VMEM((ch,), jnp.float32),
        ],
    )
    def sc_cos(memt_hbm, k_hbm, num_hbm, ssq_hbm, mem_v, k_v, num_v, ssq_v):
        b = lax.axis_index("s") * 2 + lax.axis_index("c")
        pltpu.sync_copy(k_hbm.at[b], k_v)
        kvecs = [k_v[pl.ds(mg * 16, 16)] + EPS for mg in range(M // 16)]

        @pl.loop(0, nch)
        def _chunk(c):
            n0 = n_tc + c * ch
            pltpu.sync_copy(memt_hbm.at[b, :, pl.ds(n0, ch)], mem_v)

            @pl.loop(0, ch // 16)
            def _n16(j):
                sl = pl.ds(j * 16, 16)
                accn = jnp.zeros((16,), jnp.float32)
                accs = jnp.zeros((16,), jnp.float32)
                for m in range(M):
                    v = mem_v[m, sl]
                    accn = accn + v * kvecs[m // 16][m % 16]
                    accs = accs + v * v
                num_v[sl] = accn
                ssq_v[sl] = accs

            pltpu.sync_copy(num_v, num_hbm.at[b, pl.ds(c * ch, ch)])
            pltpu.sync_copy(ssq_v, ssq_hbm.at[b, pl.ds(c * ch, ch)])

    return sc_cos


@jax.jit
def kernel(memory, k, beta, g, s, gamma, w_prev):
    B, N, M = memory.shape
    n_sc = N_SC if N > N_SC else 0
    n_tc = N - n_sc
    BN = min(2048, n_tc)
    nb = n_tc // BN
    BNr = min(2048, N)
    nbr = N // BNr
    memt = jnp.swapaxes(memory, 1, 2)

    cos_tc = pl.pallas_call(
        _cos_body,
        grid=(nb,),
        in_specs=[
            pl.BlockSpec((B, M, BN), lambda i: (0, 0, i)),
            pl.BlockSpec((B, M), lambda i: (0, 0)),
        ],
        out_specs=pl.BlockSpec((B, BN), lambda i: (0, i)),
        out_shape=jax.ShapeDtypeStruct((B, n_tc), jnp.float32),
    )(memt, k)

    if n_sc:
        num_sc, ssq_sc = _make_sc_cos(B, M, N, n_tc, SC_CHUNK)(memt, k)
    else:
        num_sc = jnp.zeros((B, 0), jnp.float32)
        ssq_sc = jnp.ones((B, 0), jnp.float32)

    w = pl.pallas_call(
        _w_body,
        in_specs=[pl.BlockSpec(x.shape, lambda: (0,) * x.ndim)
                  for x in (cos_tc, num_sc, ssq_sc, k, w_prev, beta, g, s,
                            gamma)],
        out_specs=pl.BlockSpec((B, N), lambda: (0, 0)),
        out_shape=jax.ShapeDtypeStruct((B, N), jnp.float32),
    )(cos_tc, num_sc, ssq_sc, k, w_prev, beta, g, s, gamma)

    read = pl.pallas_call(
        _read_body,
        grid=(nbr,),
        in_specs=[
            pl.BlockSpec((B, BNr), lambda i: (0, i)),
            pl.BlockSpec((B, M, BNr), lambda i: (0, 0, i)),
        ],
        out_specs=pl.BlockSpec((B, M), lambda i: (0, 0)),
        out_shape=jax.ShapeDtypeStruct((B, M), jnp.float32),
    )(w, memt)

    return read


# speedup vs baseline: 1.0569x; 1.0569x over previous
import jax
import jax.numpy as jnp
from jax.experimental import pallas as pl

EPS = 1e-16


def _cos_body(memt_ref, k_ref, cos_ref):
    memt = memt_ref[...]
    kk = k_ref[...] + EPS
    num = jnp.sum(memt * kk[:, :, None], axis=1)
    ssq = jnp.sum(memt * memt, axis=1)
    normk = jnp.sqrt(jnp.sum(kk * kk, axis=-1))
    denom = jnp.sqrt(ssq) * normk[:, None]
    cos_ref[...] = num / jnp.maximum(denom, 1e-8)


def _w_body(cos_ref, wprev_ref, beta_ref, g_ref, s_ref, gamma_ref, w_ref):
    cos = cos_ref[...]
    beta = beta_ref[...]
    x = beta * cos
    x = x - jnp.max(x, axis=1, keepdims=True)
    ex = jnp.exp(x)
    wc = ex / jnp.sum(ex, axis=1, keepdims=True)
    g = g_ref[...]
    wg = g * wc + (1.0 - g) * wprev_ref[...]
    s = s_ref[...]
    left = jnp.concatenate([wg[:, -1:], wg[:, :-1]], axis=1)
    right = jnp.concatenate([wg[:, 1:], wg[:, :1]], axis=1)
    sh = left * s[:, 0:1] + wg * s[:, 1:2] + right * s[:, 2:3]
    gamma = gamma_ref[...]
    wpow = jnp.where(sh > 0.0,
                     jnp.exp(gamma * jnp.log(jnp.maximum(sh, 1e-38))),
                     0.0)
    w_ref[...] = wpow / (jnp.sum(wpow, axis=1, keepdims=True) + EPS)


def _read_body(w_ref, memt_ref, out_ref):
    @pl.when(pl.program_id(0) == 0)
    def _():
        out_ref[...] = jnp.zeros_like(out_ref)

    w = w_ref[...]
    memt = memt_ref[...]
    out_ref[...] += jnp.sum(memt * w[:, None, :], axis=2)


@jax.jit
def kernel(memory, k, beta, g, s, gamma, w_prev):
    B, N, M = memory.shape
    BN = min(2048, N)
    nb = N // BN
    memt = jnp.swapaxes(memory, 1, 2)

    cos = pl.pallas_call(
        _cos_body,
        grid=(nb,),
        in_specs=[
            pl.BlockSpec((B, M, BN), lambda i: (0, 0, i)),
            pl.BlockSpec((B, M), lambda i: (0, 0)),
        ],
        out_specs=pl.BlockSpec((B, BN), lambda i: (0, i)),
        out_shape=jax.ShapeDtypeStruct((B, N), jnp.float32),
    )(memt, k)

    w = pl.pallas_call(
        _w_body,
        in_specs=[pl.BlockSpec(x.shape, lambda: (0,) * x.ndim)
                  for x in (cos, w_prev, beta, g, s, gamma)],
        out_specs=pl.BlockSpec((B, N), lambda: (0, 0)),
        out_shape=jax.ShapeDtypeStruct((B, N), jnp.float32),
    )(cos, w_prev, beta, g, s, gamma)

    read = pl.pallas_call(
        _read_body,
        grid=(nb,),
        in_specs=[
            pl.BlockSpec((B, BN), lambda i: (0, i)),
            pl.BlockSpec((B, M, BN), lambda i: (0, 0, i)),
        ],
        out_specs=pl.BlockSpec((B, M), lambda i: (0, 0)),
        out_shape=jax.ShapeDtypeStruct((B, M), jnp.float32),
    )(w, memt)

    return read
